# baseline (speedup 1.0000x reference)
import functools

import jax
import jax.numpy as jnp
from jax.experimental import pallas as pl
from jax.experimental.pallas import tpu as pltpu


_TILE_B = 2048


def _mlp_fused_kernel(xa_ref, xb_ref, w1a_ref, w1b_ref, b1_ref,
                      w2_ref, b2_ref, w3_ref, b3_ref,
                      w4_ref, b4_ref, o_ref):
    rhs_t = (((1,), (1,)), ((), ()))
    h = jax.lax.dot_general(xa_ref[...].astype(jnp.bfloat16), w1a_ref[...],
                            rhs_t, preferred_element_type=jnp.float32)
    h = h + jax.lax.dot_general(xb_ref[...].astype(jnp.bfloat16), w1b_ref[...],
                                rhs_t, preferred_element_type=jnp.float32)
    h = jnp.maximum(h + b1_ref[...], 0.0)
    h = jnp.dot(h, w2_ref[...], preferred_element_type=jnp.float32)
    h = jnp.maximum(h + b2_ref[...], 0.0)
    h = jax.lax.dot_general(h, w3_ref[...], rhs_t,
                            preferred_element_type=jnp.float32)
    h = jnp.maximum(h + b3_ref[...], 0.0)
    logit = jax.lax.dot_general(
        w4_ref[...], h, rhs_t,
        preferred_element_type=jnp.float32)
    o_ref[...] = jax.nn.sigmoid(logit + b4_ref[...])


@functools.partial(jax.jit, static_argnames=("tile_b",))
def _mlp_forward(x, w1, b1, w2, b2, w3, b3, w4, b4, tile_b=_TILE_B):
    batch, n_in = x.shape
    num_tiles = pl.cdiv(batch, tile_b)
    padded = num_tiles * tile_b
    if padded != batch:
        x = jnp.pad(x, ((0, padded - batch), (0, 0)))

    half = n_in // 2
    w1a = w1[:half].T.astype(jnp.bfloat16)
    w1b = w1[half:].T.astype(jnp.bfloat16)
    w3t = w3.T
    w4r = w4.T

    const = lambda i: (0, 0)
    resident = lambda a: pl.BlockSpec(a.shape, const)

    out = pl.pallas_call(
        _mlp_fused_kernel,
        out_shape=jax.ShapeDtypeStruct((1, padded), jnp.float32),
        grid=(num_tiles,),
        in_specs=[
            pl.BlockSpec((tile_b, half), lambda i: (i, 0)),
            pl.BlockSpec((tile_b, half), lambda i: (i, 1)),
            resident(w1a), resident(w1b), resident(b1),
            resident(w2), resident(b2),
            resident(w3t), resident(b3),
            resident(w4r), resident(b4),
        ],
        out_specs=pl.BlockSpec((1, tile_b), lambda i: (0, i)),
        compiler_params=pltpu.CompilerParams(
            dimension_semantics=("parallel",),
        ),
    )(x, x, w1a, w1b, b1, w2, b2, w3t, b3, w4r, b4)

    return out[0, :batch].reshape(batch, 1)


def kernel(x, w1, b1, w2, b2, w3, b3, w4, b4):
    return _mlp_forward(x, w1, b1, w2, b2, w3, b3, w4, b4)


# speedup vs baseline: 1.1509x; 1.1509x over previous
import functools

import jax
import jax.numpy as jnp
from jax.experimental import pallas as pl
from jax.experimental.pallas import tpu as pltpu


_TILE_B = 4096


def _mlp_fused_kernel(xa_ref, xb_ref, w1a_ref, w1b_ref, b1_ref,
                      w2_ref, b2_ref, w3_ref, b3_ref,
                      w4_ref, b4_ref, o_ref):
    rhs_t = (((1,), (1,)), ((), ()))
    h = jax.lax.dot_general(xa_ref[...].astype(jnp.bfloat16), w1a_ref[...],
                            rhs_t, preferred_element_type=jnp.float32)
    h = h + jax.lax.dot_general(xb_ref[...].astype(jnp.bfloat16), w1b_ref[...],
                                rhs_t, preferred_element_type=jnp.float32)
    h = jnp.maximum(h + b1_ref[...], 0.0)
    h = jnp.dot(h, w2_ref[...], preferred_element_type=jnp.float32)
    h = jnp.maximum(h + b2_ref[...], 0.0)
    h = jax.lax.dot_general(h, w3_ref[...], rhs_t,
                            preferred_element_type=jnp.float32)
    h = jnp.maximum(h + b3_ref[...], 0.0)
    logit = jax.lax.dot_general(
        w4_ref[...], h, rhs_t,
        preferred_element_type=jnp.float32)
    o_ref[...] = jax.nn.sigmoid(logit + b4_ref[...])


@functools.partial(jax.jit, static_argnames=("tile_b",))
def _mlp_forward(x, w1, b1, w2, b2, w3, b3, w4, b4, tile_b=_TILE_B):
    batch, n_in = x.shape
    num_tiles = pl.cdiv(batch, tile_b)
    padded = num_tiles * tile_b
    if padded != batch:
        x = jnp.pad(x, ((0, padded - batch), (0, 0)))

    half = n_in // 2
    w1a = w1[:half].T.astype(jnp.bfloat16)
    w1b = w1[half:].T.astype(jnp.bfloat16)
    w3t = w3.T
    w4r = w4.T

    const = lambda i: (0, 0)
    resident = lambda a: pl.BlockSpec(a.shape, const)

    out = pl.pallas_call(
        _mlp_fused_kernel,
        out_shape=jax.ShapeDtypeStruct((1, padded), jnp.float32),
        grid=(num_tiles,),
        in_specs=[
            pl.BlockSpec((tile_b, half), lambda i: (i, 0)),
            pl.BlockSpec((tile_b, half), lambda i: (i, 1)),
            resident(w1a), resident(w1b), resident(b1),
            resident(w2), resident(b2),
            resident(w3t), resident(b3),
            resident(w4r), resident(b4),
        ],
        out_specs=pl.BlockSpec((1, tile_b), lambda i: (0, i)),
        compiler_params=pltpu.CompilerParams(
            dimension_semantics=("parallel",),
        ),
    )(x, x, w1a, w1b, b1, w2, b2, w3t, b3, w4r, b4)

    return out[0, :batch].reshape(batch, 1)


def kernel(x, w1, b1, w2, b2, w3, b3, w4, b4):
    return _mlp_forward(x, w1, b1, w2, b2, w3, b3, w4, b4)


# speedup vs baseline: 1.1811x; 1.0263x over previous
import functools

import jax
import jax.numpy as jnp
from jax.experimental import pallas as pl
from jax.experimental.pallas import tpu as pltpu


_TILE_B = 4096


def _mlp_fused_kernel(xa_ref, xb_ref, w1a_ref, w1b_ref, b1_ref,
                      w2_ref, b2_ref, w3_ref, b3_ref,
                      w4_ref, b4_ref, o_ref):
    rhs_t = (((1,), (1,)), ((), ()))
    h = jax.lax.dot_general(xa_ref[...].astype(jnp.bfloat16), w1a_ref[...],
                            rhs_t, preferred_element_type=jnp.float32)
    h = h + jax.lax.dot_general(xb_ref[...].astype(jnp.bfloat16), w1b_ref[...],
                                rhs_t, preferred_element_type=jnp.float32)
    h = jnp.maximum(h + b1_ref[...], 0.0)
    h = jnp.dot(h, w2_ref[...], preferred_element_type=jnp.float32)
    h = jnp.maximum(h + b2_ref[...], 0.0)
    h = jax.lax.dot_general(h, w3_ref[...], rhs_t,
                            preferred_element_type=jnp.float32)
    h = jnp.maximum(h + b3_ref[...], 0.0)
    logit = jax.lax.dot_general(
        w4_ref[...], h, rhs_t,
        preferred_element_type=jnp.float32)
    o_ref[...] = jax.nn.sigmoid(logit + b4_ref[...])


@functools.partial(jax.jit, static_argnames=("tile_b",))
def _mlp_forward(x, w1, b1, w2, b2, w3, b3, w4, b4, tile_b=_TILE_B):
    batch, n_in = x.shape
    num_tiles = pl.cdiv(batch, tile_b)
    padded = num_tiles * tile_b
    if padded != batch:
        x = jnp.pad(x, ((0, padded - batch), (0, 0)))

    half = n_in // 2
    w1a = w1[:half].T.astype(jnp.bfloat16)
    w1b = w1[half:].T.astype(jnp.bfloat16)
    w3t = w3.T
    w4r = w4.T

    const = lambda i: (0, 0)
    resident = lambda a: pl.BlockSpec(a.shape, const)

    out = pl.pallas_call(
        _mlp_fused_kernel,
        out_shape=jax.ShapeDtypeStruct((1, padded), jnp.float32),
        grid=(num_tiles,),
        in_specs=[
            pl.BlockSpec((tile_b, half), lambda i: (i, 0)),
            pl.BlockSpec((tile_b, half), lambda i: (i, 1)),
            resident(w1a), resident(w1b), resident(b1),
            resident(w2), resident(b2),
            resident(w3t), resident(b3),
            resident(w4r), resident(b4),
        ],
        out_specs=pl.BlockSpec((1, tile_b), lambda i: (0, i)),
        compiler_params=pltpu.CompilerParams(
            dimension_semantics=("parallel",),
            allow_input_fusion=[False, False] + [True] * 8,
        ),
    )(x, x, w1a, w1b, b1, w2, b2, w3t, b3, w4r, b4)

    return out[0, :batch].reshape(batch, 1)


def kernel(x, w1, b1, w2, b2, w3, b3, w4, b4):
    return _mlp_forward(x, w1, b1, w2, b2, w3, b3, w4, b4)
